# baseline (speedup 1.0000x reference)
import jax
import jax.numpy as jnp
from jax import lax
from jax.experimental import pallas as pl
from jax.experimental.pallas import tpu as pltpu
from jax.experimental.pallas import tpu_sc as plsc

M = 1000000
D = 16
B = 16384
L = 16
NC = 2
NS = 16
NW = NC * NS
RPW = 31256
LAST = M - (NW - 1) * RPW
NV = B // L
CH = 128
NCH = B // CH


def _body(mem_hbm, idx_hbm, val_hbm, out_hbm,
          idx_v, winner, dst_l, src_l, dstq, srcq, stage, ibuf, mbuf, lmbuf,
          copy_sem, idx_sem, g_sem, s_sem):
    cid = lax.axis_index("c")
    sid = lax.axis_index("s")
    wid = sid * NC + cid
    lo = pl.multiple_of(wid * RPW, 8)
    rows = jnp.where(wid == NW - 1, LAST, RPW)

    @pl.when(wid < NW - 1)
    def _():
        pltpu.async_copy(mem_hbm.at[pl.ds(lo, RPW)],
                         out_hbm.at[pl.ds(lo, RPW)], copy_sem)

    @pl.when(wid == NW - 1)
    def _():
        pltpu.async_copy(mem_hbm.at[pl.ds(lo, LAST)],
                         out_hbm.at[pl.ds(lo, LAST)], copy_sem)

    pltpu.async_copy(idx_hbm, idx_v, idx_sem).wait()

    iota = lax.iota(jnp.int32, L)

    def scan_body(j, n):
        iv = idx_v[pl.ds(j * L, L)]
        local = iv - lo
        m = (local >= 0) & (local < rows)
        jv = j * L + iota
        plsc.store_scatter(winner, [local], jv, mask=m)
        g = plsc.load_gather(winner, [local], mask=m)
        w1 = m & (g == jv)
        has_dup = jnp.sum(m.astype(jnp.int32)) != jnp.sum(w1.astype(jnp.int32))
        lmbuf[...] = w1.astype(jnp.int32)

        @pl.when(has_dup)
        def _():
            ibuf[pl.ds(0, L)] = iv
            ibuf[pl.ds(L, L)] = jnp.full((L,), -1, jnp.int32)
            mbuf[pl.ds(0, L)] = m.astype(jnp.int32)
            mbuf[pl.ds(L, L)] = jnp.zeros((L,), jnp.int32)
            loser = jnp.zeros((L,), jnp.bool_)
            for s in range(1, L):
                shiv = ibuf[pl.ds(s, L)]
                shm = mbuf[pl.ds(s, L)] != 0
                loser = loser | ((iv == shiv) & shm)
            lmm = m & jnp.logical_not(loser)
            plsc.store_scatter(winner, [local], jv, mask=lmm)
            lmbuf[...] = lmm.astype(jnp.int32)

        lm = lmbuf[...] != 0
        plsc.store_compressed(dst_l.at[pl.ds(n, L)], local, mask=lm)
        plsc.store_compressed(src_l.at[pl.ds(n, L)], jv, mask=lm)
        return n + jnp.sum(lm.astype(jnp.int32))

    n = lax.fori_loop(0, NV, scan_body, jnp.int32(0))

    def filt_body(k, nf):
        dv = dst_l[pl.ds(k * L, L)]
        sv = src_l[pl.ds(k * L, L)]
        valid = (k * L + iota) < n
        w = plsc.load_gather(winner, [dv], mask=valid)
        keep = valid & (w == sv)
        plsc.store_compressed(dst_l.at[pl.ds(nf, L)], dv, mask=keep)
        plsc.store_compressed(src_l.at[pl.ds(nf, L)], sv, mask=keep)
        return nf + jnp.sum(keep.astype(jnp.int32))

    nf = lax.fori_loop(0, (n + L - 1) // L, filt_body, jnp.int32(0))

    nch = (nf + CH - 1) // CH

    @pl.when(nf > 0)
    def _():
        zeros = jnp.zeros((L,), jnp.int32)
        d0 = plsc.load_gather(dst_l, [zeros])
        s0 = plsc.load_gather(src_l, [zeros])

        def pack_body(t, _):
            dv = dst_l[pl.ds(t * L, L)]
            sv = src_l[pl.ds(t * L, L)]
            mvalid = (t * L + iota) < nf
            dglob = jnp.where(mvalid, dv, d0) + lo
            sfix = jnp.where(mvalid, sv, s0)
            c = t // (CH // L)
            r = (t % (CH // L)) * L
            dstq.at[c][pl.ds(r, L)] = dglob
            srcq.at[c][pl.ds(r, L)] = sfix
            return 0

        lax.fori_loop(0, nch * (CH // L), pack_body, 0)

    @pl.when(wid < NW - 1)
    def _():
        pltpu.make_async_copy(mem_hbm.at[pl.ds(lo, RPW)],
                              out_hbm.at[pl.ds(lo, RPW)], copy_sem).wait()

    @pl.when(wid == NW - 1)
    def _():
        pltpu.make_async_copy(mem_hbm.at[pl.ds(lo, LAST)],
                              out_hbm.at[pl.ds(lo, LAST)], copy_sem).wait()

    def chunk_body(c, _):
        pltpu.async_copy(val_hbm.at[srcq.at[c]], stage, g_sem).wait()
        pltpu.async_copy(stage, out_hbm.at[dstq.at[c]], s_sem).wait()
        return 0

    lax.fori_loop(0, nch, chunk_body, 0)


_scatter_call = pl.kernel(
    _body,
    out_type=jax.ShapeDtypeStruct((M, D), jnp.float32),
    mesh=plsc.VectorSubcoreMesh(core_axis_name="c", subcore_axis_name="s"),
    compiler_params=pltpu.CompilerParams(needs_layout_passes=False,
                                         use_tc_tiling_on_sc=False),
    scratch_types=[
        pltpu.VMEM((B,), jnp.int32),
        pltpu.VMEM((RPW,), jnp.int32),
        pltpu.VMEM((B + L,), jnp.int32),
        pltpu.VMEM((B + L,), jnp.int32),
        pltpu.VMEM((NCH, CH), jnp.int32),
        pltpu.VMEM((NCH, CH), jnp.int32),
        pltpu.VMEM((CH, D), jnp.float32),
        pltpu.VMEM((2 * L,), jnp.int32),
        pltpu.VMEM((2 * L,), jnp.int32),
        pltpu.VMEM((L,), jnp.int32),
        pltpu.SemaphoreType.DMA,
        pltpu.SemaphoreType.DMA,
        pltpu.SemaphoreType.DMA,
        pltpu.SemaphoreType.DMA,
    ],
)


def kernel(mem, idx, val):
    return _scatter_call(mem, idx.astype(jnp.int32), val)


# speedup vs baseline: 2.8221x; 2.8221x over previous
import jax
import jax.numpy as jnp
from jax import lax
from jax.experimental import pallas as pl
from jax.experimental.pallas import tpu as pltpu
from jax.experimental.pallas import tpu_sc as plsc

M = 1000000
D = 16
B = 16384
L = 16
NC = 2
NS = 16
NW = NC * NS
RPW = 31256
LAST = M - (NW - 1) * RPW
NV = B // L
CH = 128
CPR = 1024
NCP = 31
SPC = 34


def _body(mem_hbm, idx_hbm, val_hbm, out_hbm,
          idx_v, winner, dst_l, src_l, dstrow, srcrow, stage,
          buf0, buf1, ibuf, mbuf, lmbuf, cbuf,
          in_sem0, in_sem1, out_sem0, out_sem1, idx_sem, g_sem, s_sem):
    cid = lax.axis_index("c")
    sid = lax.axis_index("s")
    wid = sid * NC + cid
    lo = pl.multiple_of(wid * RPW, 8)
    rows = jnp.where(wid == NW - 1, LAST, RPW)

    def cbase(c):
        return pl.multiple_of(jnp.minimum(c * CPR, rows - CPR), 8)

    pltpu.async_copy(idx_hbm, idx_v, idx_sem).wait()

    iota = lax.iota(jnp.int32, L)

    def scan_body(j, n):
        iv = idx_v[pl.ds(j * L, L)]
        local = iv - lo
        m = (local >= 0) & (local < rows)
        nm = jnp.sum(m.astype(jnp.int32))
        jv = j * L + iota
        cbuf[0] = jnp.int32(0)

        @pl.when(nm > 0)
        def _():
            plsc.store_scatter(winner, [local], jv, mask=m)
            g = plsc.load_gather(winner, [local], mask=m)
            w1 = m & (g == jv)
            lmbuf[...] = w1.astype(jnp.int32)
            cbuf[0] = nm

            @pl.when(nm != jnp.sum(w1.astype(jnp.int32)))
            def _():
                ibuf[pl.ds(0, L)] = iv
                ibuf[pl.ds(L, L)] = jnp.full((L,), -1, jnp.int32)
                mbuf[pl.ds(0, L)] = m.astype(jnp.int32)
                mbuf[pl.ds(L, L)] = jnp.zeros((L,), jnp.int32)
                loser = jnp.zeros((L,), jnp.bool_)
                for s in range(1, L):
                    shiv = ibuf[pl.ds(s, L)]
                    shm = mbuf[pl.ds(s, L)] != 0
                    loser = loser | ((iv == shiv) & shm)
                lmm = m & jnp.logical_not(loser)
                plsc.store_scatter(winner, [local], jv, mask=lmm)
                lmbuf[...] = lmm.astype(jnp.int32)
                cbuf[0] = jnp.sum(lmm.astype(jnp.int32))

            lm = lmbuf[...] != 0
            plsc.store_compressed(dst_l.at[pl.ds(n, L)], local, mask=lm)
            plsc.store_compressed(src_l.at[pl.ds(n, L)], jv, mask=lm)

        return n + cbuf[0]

    bufs = ((buf0, in_sem0, out_sem0), (buf1, in_sem1, out_sem1))
    for p, (buf, isem, _) in enumerate(bufs):
        pltpu.async_copy(mem_hbm.at[pl.ds(cbase(p), CPR)], buf, isem)

    def copy_chunk(c, n):
        b = cbase(c)
        for p, (buf, isem, osem) in enumerate(bufs):
            @pl.when((c & 1) == p)
            def _():
                pltpu.make_async_copy(mem_hbm.at[pl.ds(b, CPR)], buf,
                                      isem).wait()
                pltpu.async_copy(buf, out_hbm.at[pl.ds(b, CPR)], osem)

        n = lax.fori_loop(c * SPC, jnp.minimum((c + 1) * SPC, NV),
                          scan_body, n)

        for p, (buf, isem, osem) in enumerate(bufs):
            @pl.when((c & 1) == p)
            def _():
                pltpu.make_async_copy(buf, out_hbm.at[pl.ds(b, CPR)],
                                      osem).wait()

                @pl.when(c + 2 < NCP)
                def _():
                    pltpu.async_copy(mem_hbm.at[pl.ds(cbase(c + 2), CPR)],
                                     buf, isem)

        return n

    n = lax.fori_loop(0, NCP, copy_chunk, jnp.int32(0))

    def filt_body(k, nf):
        dv = dst_l[pl.ds(k * L, L)]
        sv = src_l[pl.ds(k * L, L)]
        valid = (k * L + iota) < n
        w = plsc.load_gather(winner, [dv], mask=valid)
        keep = valid & (w == sv)
        plsc.store_compressed(dst_l.at[pl.ds(nf, L)], dv, mask=keep)
        plsc.store_compressed(src_l.at[pl.ds(nf, L)], sv, mask=keep)
        return nf + jnp.sum(keep.astype(jnp.int32))

    nf = lax.fori_loop(0, (n + L - 1) // L, filt_body, jnp.int32(0))

    @pl.when(nf > 0)
    def _():
        zeros16 = jnp.zeros((L,), jnp.int32)
        d0 = plsc.load_gather(dst_l, [zeros16])
        s0 = plsc.load_gather(src_l, [zeros16])
        nch = (nf + CH - 1) // CH

        def sc_chunk(c, _):
            for t in range(CH // L):
                off = c * CH + t * L
                dv = dst_l[pl.ds(off, L)]
                sv = src_l[pl.ds(off, L)]
                mv = (off + iota) < nf
                dstrow.at[0][pl.ds(t * L, L)] = jnp.where(mv, dv, d0) + lo
                srcrow.at[0][pl.ds(t * L, L)] = jnp.where(mv, sv, s0)
            pltpu.async_copy(val_hbm.at[srcrow.at[0]], stage, g_sem).wait()
            pltpu.async_copy(stage, out_hbm.at[dstrow.at[0]], s_sem).wait()
            return 0

        lax.fori_loop(0, nch, sc_chunk, 0)


_scatter_call = pl.kernel(
    _body,
    out_type=jax.ShapeDtypeStruct((M, D), jnp.float32),
    mesh=plsc.VectorSubcoreMesh(core_axis_name="c", subcore_axis_name="s"),
    compiler_params=pltpu.CompilerParams(needs_layout_passes=False,
                                         use_tc_tiling_on_sc=False),
    scratch_types=[
        pltpu.VMEM((B,), jnp.int32),
        pltpu.VMEM((RPW,), jnp.int32),
        pltpu.VMEM((B + L,), jnp.int32),
        pltpu.VMEM((B + L,), jnp.int32),
        pltpu.VMEM((1, CH), jnp.int32),
        pltpu.VMEM((1, CH), jnp.int32),
        pltpu.VMEM((CH, D), jnp.float32),
        pltpu.VMEM((CPR, D), jnp.float32),
        pltpu.VMEM((CPR, D), jnp.float32),
        pltpu.VMEM((2 * L,), jnp.int32),
        pltpu.VMEM((2 * L,), jnp.int32),
        pltpu.VMEM((L,), jnp.int32),
        pltpu.SMEM((1,), jnp.int32),
        pltpu.SemaphoreType.DMA,
        pltpu.SemaphoreType.DMA,
        pltpu.SemaphoreType.DMA,
        pltpu.SemaphoreType.DMA,
        pltpu.SemaphoreType.DMA,
        pltpu.SemaphoreType.DMA,
        pltpu.SemaphoreType.DMA,
    ],
)


def kernel(mem, idx, val):
    return _scatter_call(mem, idx.astype(jnp.int32), val)
